# Initial kernel scaffold; baseline (speedup 1.0000x reference)
import jax
import jax.numpy as jnp
from jax.experimental import pallas as pl


def kernel(x, xe, ye, I, log_temp):
    raise NotImplementedError("write your pallas kernel here")


# speedup vs baseline: 4.1606x; 4.1606x over previous
import functools
import math

import jax
import jax.numpy as jnp
from jax import lax
from jax.experimental import pallas as pl
from jax.experimental.pallas import tpu as pltpu
from jax.experimental.pallas import tpu_sc as plsc

B, N, F, M, O, KS = 4, 4096, 64, 1024, 16, 4
BM = B * M
NW = 32
QPW = BM // NW
QCH = 8
NCH = QPW // QCH
LOG_HALF = math.log(0.5)


def _sc_gather(tab, iflat):
    mesh = plsc.VectorSubcoreMesh(core_axis_name="c", subcore_axis_name="s")

    @functools.partial(
        pl.kernel,
        mesh=mesh,
        out_type=jax.ShapeDtypeStruct((BM * O, 2 * F), jnp.float32),
        scratch_types=(
            pltpu.VMEM((QCH * O,), jnp.int32),
            pltpu.VMEM((QCH * O, 2 * F), jnp.float32),
            pltpu.SemaphoreType.DMA,
        ),
    )
    def k(tab_hbm, i_hbm, g_hbm, idx_v, g_v, sem):
        wid = lax.axis_index("s") * 2 + lax.axis_index("c")
        row_off = (wid // (M // QPW)) * N
        for c in range(NCH):
            ibase = (wid * QPW + c * QCH) * O
            pltpu.sync_copy(i_hbm.at[pl.ds(ibase, QCH * O)], idx_v)
            for t in range(QCH):
                sl = pl.ds(t * O, O)
                idx_v[sl] = idx_v[sl] + row_off
            pltpu.async_copy(tab_hbm.at[idx_v], g_v, sem).wait()
            pltpu.sync_copy(g_v, g_hbm.at[pl.ds(ibase, QCH * O)])

    return k(tab, iflat)


BMBLK = 512
DBLK = 256


def _sum64(v):
    p = v[:, 0:8]
    for i in range(1, 8):
        p = p + v[:, 8 * i : 8 * i + 8]
    a = p[:, :4] + p[:, 4:]
    b = a[:, :2] + a[:, 2:]
    return b[:, :1] + b[:, 1:]


def _dist_body(g_ref, ye_ref, d_ref):
    xg = g_ref[...][:, F:]
    ye = ye_ref[...]
    mm = lax.dot_general(ye, xg, (((1,), (1,)), ((), ())))
    mm3 = mm.reshape(DBLK, DBLK, O)
    qi = lax.broadcasted_iota(jnp.int32, (DBLK, DBLK, 1), 0)
    pi = lax.broadcasted_iota(jnp.int32, (DBLK, DBLK, 1), 1)
    dot = jnp.sum(jnp.where(qi == pi, mm3, 0.0), axis=1)
    nxe = _sum64(xg * xg).reshape(DBLK, O)
    nye = _sum64(ye * ye)
    d_ref[...] = -((-2.0 * dot + nxe) + nye)


def _tc_dist(g, ye2):
    return pl.pallas_call(
        _dist_body,
        grid=(BM // DBLK,),
        in_specs=[
            pl.BlockSpec((DBLK * O, 2 * F), lambda i: (i, 0)),
            pl.BlockSpec((DBLK, F), lambda i: (i, 0)),
        ],
        out_specs=pl.BlockSpec((DBLK, O), lambda i: (i, 0)),
        out_shape=jax.ShapeDtypeStruct((BM, O), jnp.float32),
    )(g, ye2)


def _agg_body(g_ref, w0_ref, w1_ref, w2_ref, w3_ref, *z_refs):
    x3 = g_ref[...].reshape(BMBLK, O, 2 * F)[:, :, :F]
    for k, w_ref in enumerate((w0_ref, w1_ref, w2_ref, w3_ref)):
        z_refs[k][...] = jnp.sum(w_ref[...][:, :, None] * x3, axis=1)


def _tc_agg(g, ws):
    return pl.pallas_call(
        _agg_body,
        grid=(BM // BMBLK,),
        in_specs=[pl.BlockSpec((BMBLK * O, 2 * F), lambda i: (i, 0))]
        + [pl.BlockSpec((BMBLK, O), lambda i: (i, 0))] * KS,
        out_specs=[pl.BlockSpec((BMBLK, F), lambda i: (i, 0))] * KS,
        out_shape=[jax.ShapeDtypeStruct((BM, F), jnp.float32)] * KS,
    )(g, *ws)


def _log1mexp_ref(x, guard=1e-07):
    t = x < LOG_HALF
    xs = jnp.where(t, x, -1.0)
    xl = jnp.where(t, -1.0, x)
    ys = jnp.log1p(-jnp.exp(xs))
    em = -jnp.expm1(xl)
    fw = jnp.log(em)
    bw = jnp.log(em + guard)
    yl = jax.lax.stop_gradient(fw) + (bw - jax.lax.stop_gradient(bw))
    return jnp.where(t, ys, yl)


def kernel(x, xe, ye, I, log_temp):
    tab = jnp.concatenate([x.reshape(B * N, F), xe.reshape(B * N, F)], axis=1)
    iflat = I.astype(jnp.int32).reshape(BM * O)
    g = _sc_gather(tab, iflat)
    ye2 = ye.reshape(BM, F)
    D = _tc_dist(g, ye2)

    temperature = jnp.exp(log_temp[0])
    logits = D / temperature
    ws = []
    for r in range(KS):
        w = jax.nn.log_softmax(logits, axis=1)
        ws.append(jnp.exp(w))
        if r < KS - 1:
            logits = logits + _log1mexp_ref(w)

    z0, z1, z2, z3 = _tc_agg(g, ws)
    z = jnp.stack([z0, z1, z2, z3], axis=-1)
    return z.reshape(B, M, F, KS)


# speedup vs baseline: 4.3952x; 1.0564x over previous
import functools
import math

import jax
import jax.numpy as jnp
from jax import lax
from jax.experimental import pallas as pl
from jax.experimental.pallas import tpu as pltpu
from jax.experimental.pallas import tpu_sc as plsc

B, N, F, M, O, KS = 4, 4096, 64, 1024, 16, 4
BM = B * M
NW = 32
QPW = BM // NW
QCH = 8
NCH = QPW // QCH
LOG_HALF = math.log(0.5)


def _sc_gather(tab, iflat):
    mesh = plsc.VectorSubcoreMesh(core_axis_name="c", subcore_axis_name="s")

    @functools.partial(
        pl.kernel,
        mesh=mesh,
        out_type=jax.ShapeDtypeStruct((BM * O, 2 * F), jnp.float32),
        scratch_types=(
            pltpu.VMEM((QCH * O,), jnp.int32),
            pltpu.VMEM((QCH * O,), jnp.int32),
            pltpu.VMEM((QCH * O, 2 * F), jnp.float32),
            pltpu.VMEM((QCH * O, 2 * F), jnp.float32),
            pltpu.SemaphoreType.DMA,
            pltpu.SemaphoreType.DMA,
        ),
    )
    def k(tab_hbm, i_hbm, g_hbm, idx0, idx1, g0, g1, sem0, sem1):
        wid = lax.axis_index("s") * 2 + lax.axis_index("c")
        row_off = (wid // (M // QPW)) * N
        idx_v = (idx0, idx1)
        g_v = (g0, g1)
        sems = (sem0, sem1)

        def issue(c):
            sl = c % 2
            ibase = (wid * QPW + c * QCH) * O
            pltpu.sync_copy(i_hbm.at[pl.ds(ibase, QCH * O)], idx_v[sl])
            for t in range(QCH):
                s = pl.ds(t * O, O)
                idx_v[sl][s] = idx_v[sl][s] + row_off
            return pltpu.async_copy(tab_hbm.at[idx_v[sl]], g_v[sl], sems[sl])

        cp = issue(0)
        for c in range(NCH):
            nxt = issue(c + 1) if c + 1 < NCH else None
            cp.wait()
            ibase = (wid * QPW + c * QCH) * O
            pltpu.sync_copy(g_v[c % 2], g_hbm.at[pl.ds(ibase, QCH * O)])
            cp = nxt

    return k(tab, iflat)


BMBLK = 512
DBLK = 256


def _sum64(v):
    p = v[:, 0:8]
    for i in range(1, 8):
        p = p + v[:, 8 * i : 8 * i + 8]
    a = p[:, :4] + p[:, 4:]
    b = a[:, :2] + a[:, 2:]
    return b[:, :1] + b[:, 1:]


def _dist_body(g_ref, ye_ref, d_ref):
    xg = g_ref[...][:, F:]
    ye = ye_ref[...]
    mm = lax.dot_general(ye, xg, (((1,), (1,)), ((), ())))
    mm3 = mm.reshape(DBLK, DBLK, O)
    qi = lax.broadcasted_iota(jnp.int32, (DBLK, DBLK, 1), 0)
    pi = lax.broadcasted_iota(jnp.int32, (DBLK, DBLK, 1), 1)
    dot = jnp.sum(jnp.where(qi == pi, mm3, 0.0), axis=1)
    nxe = _sum64(xg * xg).reshape(DBLK, O)
    nye = _sum64(ye * ye)
    d_ref[...] = -((-2.0 * dot + nxe) + nye)


def _tc_dist(g, ye2):
    return pl.pallas_call(
        _dist_body,
        grid=(BM // DBLK,),
        in_specs=[
            pl.BlockSpec((DBLK * O, 2 * F), lambda i: (i, 0)),
            pl.BlockSpec((DBLK, F), lambda i: (i, 0)),
        ],
        out_specs=pl.BlockSpec((DBLK, O), lambda i: (i, 0)),
        out_shape=jax.ShapeDtypeStruct((BM, O), jnp.float32),
    )(g, ye2)


def _agg_body(g_ref, w0_ref, w1_ref, w2_ref, w3_ref, *z_refs):
    x3 = g_ref[...].reshape(BMBLK, O, 2 * F)[:, :, :F]
    for k, w_ref in enumerate((w0_ref, w1_ref, w2_ref, w3_ref)):
        z_refs[k][...] = jnp.sum(w_ref[...][:, :, None] * x3, axis=1)


def _tc_agg(g, ws):
    return pl.pallas_call(
        _agg_body,
        grid=(BM // BMBLK,),
        in_specs=[pl.BlockSpec((BMBLK * O, 2 * F), lambda i: (i, 0))]
        + [pl.BlockSpec((BMBLK, O), lambda i: (i, 0))] * KS,
        out_specs=[pl.BlockSpec((BMBLK, F), lambda i: (i, 0))] * KS,
        out_shape=[jax.ShapeDtypeStruct((BM, F), jnp.float32)] * KS,
    )(g, *ws)


def _log1mexp_ref(x, guard=1e-07):
    t = x < LOG_HALF
    xs = jnp.where(t, x, -1.0)
    xl = jnp.where(t, -1.0, x)
    ys = jnp.log1p(-jnp.exp(xs))
    em = -jnp.expm1(xl)
    fw = jnp.log(em)
    bw = jnp.log(em + guard)
    yl = jax.lax.stop_gradient(fw) + (bw - jax.lax.stop_gradient(bw))
    return jnp.where(t, ys, yl)


def kernel(x, xe, ye, I, log_temp):
    tab = jnp.concatenate([x.reshape(B * N, F), xe.reshape(B * N, F)], axis=1)
    iflat = I.astype(jnp.int32).reshape(BM * O)
    g = _sc_gather(tab, iflat)
    ye2 = ye.reshape(BM, F)
    D = _tc_dist(g, ye2)

    temperature = jnp.exp(log_temp[0])
    logits = D / temperature
    ws = []
    for r in range(KS):
        w = jax.nn.log_softmax(logits, axis=1)
        ws.append(jnp.exp(w))
        if r < KS - 1:
            logits = logits + _log1mexp_ref(w)

    z0, z1, z2, z3 = _tc_agg(g, ws)
    z = jnp.stack([z0, z1, z2, z3], axis=-1)
    return z.reshape(B, M, F, KS)
